# baseline (speedup 1.0000x reference)
import jax
import jax.numpy as jnp
from jax import lax
from jax.experimental import pallas as pl
from jax.experimental.pallas import tpu as pltpu
from jax.experimental.pallas import tpu_sc as plsc

B = 128
V = 100000
K = 256

NC = 2
NS = 16
L = 16
NW = NC * NS
CV = 128
NCH_FULL = V // CV
TAIL_V = V - NCH_FULL * CV
NKMAIN = 24
W_EXTRA = NCH_FULL - NKMAIN * NW
CAP = B * K
PAD = 12 * L
PIECE = 16
SENT = (NCH_FULL + 1) * CV * B
RUN = 4


def kernel(input_ids, scores, allowed_idx):
    del input_ids

    mesh = plsc.VectorSubcoreMesh(
        core_axis_name="c", subcore_axis_name="s", num_cores=NC,
        num_subcores=NS)

    @pl.kernel(
        out_type=jax.ShapeDtypeStruct((V, B), jnp.float32),
        mesh=mesh,
        compiler_params=pltpu.CompilerParams(needs_layout_passes=False),
        scratch_types=[
            pltpu.VMEM((CAP + PAD,), jnp.int32),
            pltpu.VMEM((CAP + PAD,), jnp.float32),
            pltpu.VMEM((2, CV, B), jnp.float32),
            pltpu.VMEM((128, B), jnp.float32),
            pltpu.VMEM((PIECE, K), jnp.int32),
            pltpu.VMEM((128,), jnp.int32),
            pltpu.SemaphoreType.DMA,
            pltpu.SemaphoreType.DMA,
            pltpu.SemaphoreType.DMA,
        ],
    )
    def sc_kernel(scores_hbm, idx_hbm, out_hbm, flist, vals, out_v, stage,
                  piece, vrow, gsem, osem0, osem1):
        w = lax.axis_index("s") * NC + lax.axis_index("c")
        neg_inf = jnp.full((L,), -jnp.inf, jnp.float32)
        sent16 = jnp.full((L,), SENT, jnp.int32)
        iota = lax.iota(jnp.int32, L)
        osems = (osem0, osem1)

        with jax.named_scope("ph_fill"):
            for bi in range(2):
                def fill(r, carry, bi=bi):
                    for u in range(B // L):
                        out_v[bi, r, pl.ds(u * L, L)] = neg_inf
                    return carry
                lax.fori_loop(0, CV, fill, 0)

        def scan_row(r, cnt, p):
            b_base = p * PIECE
            for j in range(K // L):
                iv = piece[r, pl.ds(j * L, L)]
                f = iv * B + (b_base + r)
                m = ((f >> 14) & (NW - 1)) == w
                plsc.store_compressed(flist.at[pl.ds(cnt, L)], f, mask=m)
                cnt = cnt + plsc.all_reduce_population_count(m)[0]
            return cnt

        with jax.named_scope("ph_scan"):
            cnt = jnp.int32(0)
            for p in range(B // PIECE):
                pltpu.sync_copy(idx_hbm.at[pl.ds(p * PIECE, PIECE)], piece)
                cnt = lax.fori_loop(
                    0, PIECE, lambda r, c, p=p: scan_row(r, c, p), cnt)

            for j in range(PAD // L):
                flist[pl.ds(cnt + j * L, L)] = sent16

        def sb_body(sb, carry):
            base = sb * 128
            for j in range(128 // L):
                fv = flist[pl.ds(base + j * L, L)]
                vrow[pl.ds(j * L, L)] = jnp.minimum(fv >> 7, V - 1)
            pltpu.async_copy(scores_hbm.at[vrow], stage, gsem).wait()
            for j in range(128 // L):
                fv = flist[pl.ds(base + j * L, L)]
                rowv = iota + j * L
                vv = plsc.load_gather(stage, [rowv, fv & (B - 1)])
                vals[pl.ds(base + j * L, L)] = vv
            return carry

        with jax.named_scope("ph_gather"):
            nsb = (cnt + 127) >> 7
            lax.fori_loop(0, nsb, sb_body, 0)

        n4 = (cnt + RUN * L - 1) >> 6

        def chunk_pass(c, bi, restore):
            cb = c * CV

            def body(i, carry):
                for u in range(RUN):
                    o = (i * RUN + u) * L
                    fv = flist[pl.ds(o, L)]
                    m = (fv >> 14) == c
                    lv = (fv >> 7) - cb
                    bv = fv & (B - 1)
                    x = neg_inf if restore else vals[pl.ds(o, L)]
                    plsc.store_scatter(out_v.at[bi], [lv, bv], x, mask=m)
                return carry

            lax.fori_loop(0, n4, body, 0)

        def issue_out(bi, c):
            lo = pl.multiple_of(c * CV, CV)
            return pltpu.async_copy(
                out_v.at[bi], out_hbm.at[pl.ds(lo, CV)], osems[bi])

        def wait_out(bi, c):
            lo = pl.multiple_of(c * CV, CV)
            pltpu.make_async_copy(
                out_v.at[bi], out_hbm.at[pl.ds(lo, CV)], osems[bi]).wait()

        def pair(g, carry):
            for bi in range(2):
                k = 2 * g + bi
                c = w + NW * k

                @pl.when(g > 0)
                def _():
                    wait_out(bi, c)
                    chunk_pass(c - 2 * NW, bi, restore=True)

                chunk_pass(c, bi, restore=False)
                issue_out(bi, c)
            return carry

        with jax.named_scope("ph_chunks"):
            lax.fori_loop(0, NKMAIN // 2, pair, 0)

        c22 = w + NW * (NKMAIN - 2)
        wait_out(0, c22)
        chunk_pass(c22, 0, restore=True)

        @pl.when(w < W_EXTRA)
        def _():
            c24 = w + NW * NKMAIN
            chunk_pass(c24, 0, restore=False)
            issue_out(0, c24)
            wait_out(0, c24)

        @pl.when(w == W_EXTRA)
        def _():
            tb = NCH_FULL * CV

            def body(i, carry):
                for u in range(RUN):
                    o = (i * RUN + u) * L
                    fv = flist[pl.ds(o, L)]
                    m = (fv >> 14) == NCH_FULL
                    lv = (fv >> 7) - tb
                    bv = fv & (B - 1)
                    vv = vals[pl.ds(o, L)]
                    plsc.store_scatter(out_v.at[0], [lv, bv], vv, mask=m)
                return carry

            lax.fori_loop(0, n4, body, 0)
            pltpu.async_copy(
                out_v.at[0].at[pl.ds(0, TAIL_V)],
                out_hbm.at[pl.ds(tb, TAIL_V)], osem0).wait()

        wait_out(1, w + NW * (NKMAIN - 1))

    out_t = sc_kernel(scores.T, allowed_idx)
    return out_t.T


# speedup vs baseline: 2.5244x; 2.5244x over previous
import jax
import jax.numpy as jnp
from jax import lax
from jax.experimental import pallas as pl
from jax.experimental.pallas import tpu as pltpu
from jax.experimental.pallas import tpu_sc as plsc

B = 128
V = 100000
K = 256

NC = 2
NS = 16
L = 16
NW = NC * NS
CV = 128
NCH_FULL = V // CV
TAIL_V = V - NCH_FULL * CV
NKMAIN = 24
W_EXTRA = NCH_FULL - NKMAIN * NW
CAP = B * K
PAD = 12 * L
PIECE = 16
SENT = (NCH_FULL + 1) * CV * B
RUN = 4


def kernel(input_ids, scores, allowed_idx):
    del input_ids

    mesh = plsc.VectorSubcoreMesh(
        core_axis_name="c", subcore_axis_name="s", num_cores=NC,
        num_subcores=NS)

    @pl.kernel(
        out_type=jax.ShapeDtypeStruct((V, B), jnp.float32),
        mesh=mesh,
        compiler_params=pltpu.CompilerParams(needs_layout_passes=False),
        scratch_types=[
            pltpu.VMEM((CAP + PAD,), jnp.int32),
            pltpu.VMEM((2, CV, B), jnp.float32),
            pltpu.VMEM((2, CV, B), jnp.float32),
            pltpu.VMEM((PIECE, K), jnp.int32),
            pltpu.SemaphoreType.DMA,
            pltpu.SemaphoreType.DMA,
            pltpu.SemaphoreType.DMA,
            pltpu.SemaphoreType.DMA,
        ],
    )
    def sc_kernel(scores_hbm, idx_hbm, out_hbm, flist, in_v, out_v,
                  piece, isem0, isem1, osem0, osem1):
        w = lax.axis_index("s") * NC + lax.axis_index("c")
        neg_inf = jnp.full((L,), -jnp.inf, jnp.float32)
        sent16 = jnp.full((L,), SENT, jnp.int32)
        isems = (isem0, isem1)
        osems = (osem0, osem1)

        def chunk_of(k):
            return w + NW * k

        def issue_in(bi, c):
            lo = pl.multiple_of(c * CV, CV)
            return pltpu.async_copy(
                scores_hbm.at[pl.ds(lo, CV)], in_v.at[bi], isems[bi])

        def wait_in(bi, c):
            lo = pl.multiple_of(c * CV, CV)
            pltpu.make_async_copy(
                scores_hbm.at[pl.ds(lo, CV)], in_v.at[bi], isems[bi]).wait()

        def issue_out(bi, c):
            lo = pl.multiple_of(c * CV, CV)
            return pltpu.async_copy(
                out_v.at[bi], out_hbm.at[pl.ds(lo, CV)], osems[bi])

        def wait_out(bi, c):
            lo = pl.multiple_of(c * CV, CV)
            pltpu.make_async_copy(
                out_v.at[bi], out_hbm.at[pl.ds(lo, CV)], osems[bi]).wait()

        issue_in(0, chunk_of(0))
        issue_in(1, chunk_of(1))

        for bi in range(2):
            def fill(r, carry, bi=bi):
                for u in range(B // L):
                    out_v[bi, r, pl.ds(u * L, L)] = neg_inf
                return carry
            lax.fori_loop(0, CV, fill, 0)

        def scan_row(r, cnt, p):
            b_base = p * PIECE
            for j in range(K // L):
                iv = piece[r, pl.ds(j * L, L)]
                f = iv * B + (b_base + r)
                m = ((f >> 14) & (NW - 1)) == w
                plsc.store_compressed(flist.at[pl.ds(cnt, L)], f, mask=m)
                cnt = cnt + plsc.all_reduce_population_count(m)[0]
            return cnt

        cnt = jnp.int32(0)
        for p in range(B // PIECE):
            pltpu.sync_copy(idx_hbm.at[pl.ds(p * PIECE, PIECE)], piece)
            cnt = lax.fori_loop(
                0, PIECE, lambda r, c, p=p: scan_row(r, c, p), cnt)

        for j in range(PAD // L):
            flist[pl.ds(cnt + j * L, L)] = sent16

        n4 = (cnt + RUN * L - 1) >> 6

        def chunk_pass(c, bi, restore):
            cb = c * CV

            def body(i, carry):
                for u in range(RUN):
                    o = (i * RUN + u) * L
                    fv = flist[pl.ds(o, L)]
                    m = (fv >> 14) == c
                    lv = (fv >> 7) - cb
                    bv = fv & (B - 1)
                    if restore:
                        x = neg_inf
                    else:
                        x = plsc.load_gather(in_v.at[bi], [lv, bv], mask=m)
                    plsc.store_scatter(out_v.at[bi], [lv, bv], x, mask=m)
                return carry

            lax.fori_loop(0, n4, body, 0)

        def pair(g, carry):
            for bi in range(2):
                k = 2 * g + bi
                c = chunk_of(k)

                @pl.when(g > 0)
                def _():
                    wait_out(bi, c)
                    chunk_pass(c - 2 * NW, bi, restore=True)

                wait_in(bi, c)
                chunk_pass(c, bi, restore=False)
                issue_out(bi, c)

                @pl.when(g < NKMAIN // 2 - 1)
                def _():
                    issue_in(bi, c + 2 * NW)
            return carry

        lax.fori_loop(0, NKMAIN // 2, pair, 0)

        c22 = chunk_of(NKMAIN - 2)
        wait_out(0, c22)
        chunk_pass(c22, 0, restore=True)

        @pl.when(w < W_EXTRA)
        def _():
            c24 = chunk_of(NKMAIN)
            issue_in(0, c24)
            wait_in(0, c24)
            chunk_pass(c24, 0, restore=False)
            issue_out(0, c24)
            wait_out(0, c24)

        @pl.when(w == W_EXTRA)
        def _():
            tb = NCH_FULL * CV
            pltpu.sync_copy(
                scores_hbm.at[pl.ds(tb, TAIL_V)],
                in_v.at[0].at[pl.ds(0, TAIL_V)])

            def body(i, carry):
                for u in range(RUN):
                    o = (i * RUN + u) * L
                    fv = flist[pl.ds(o, L)]
                    m = (fv >> 14) == NCH_FULL
                    lv = (fv >> 7) - tb
                    bv = fv & (B - 1)
                    vv = plsc.load_gather(in_v.at[0], [lv, bv], mask=m)
                    plsc.store_scatter(out_v.at[0], [lv, bv], vv, mask=m)
                return carry

            lax.fori_loop(0, n4, body, 0)
            pltpu.async_copy(
                out_v.at[0].at[pl.ds(0, TAIL_V)],
                out_hbm.at[pl.ds(tb, TAIL_V)], osem0).wait()

        wait_out(1, chunk_of(NKMAIN - 1))

    out_t = sc_kernel(scores.T, allowed_idx)
    return out_t.T
